# Initial kernel scaffold; baseline (speedup 1.0000x reference)
import functools

import jax
import jax.numpy as jnp
from jax import lax
from jax.experimental import pallas as pl
from jax.experimental.pallas import tpu as pltpu
from jax.experimental.pallas import tpu_sc as plsc

N = 10000
H = 128
L = 3
NG = 16
E = 300000

NC = 2
NS = 16
SUB = 4
CHUNK = SUB * 128
BPT = -(-E // (NS * CHUNK))
EPT = BPT * CHUNK
EPAD = NS * EPT
ROWS_PT = 626
NPAD = NS * ROWS_PT
BLK = 1000
GRID = N // BLK

_sc_mesh = plsc.VectorSubcoreMesh(
    core_axis_name="c", subcore_axis_name="s", num_cores=NC, num_subcores=NS)


@functools.partial(
    pl.kernel,
    out_type=jax.ShapeDtypeStruct((NC * NPAD, 16), jnp.float32),
    mesh=_sc_mesh,
    scratch_types=[
        pltpu.VMEM((SUB, 128), jnp.int32),
        pltpu.VMEM((128, 16), jnp.float32),
        pltpu.VMEM((512, 16), jnp.float32),
        pltpu.VMEM_SHARED((NPAD, 16), jnp.float32),
    ],
)
def _sc_counts(dst_hbm, ones_hbm, zeros_hbm, out_hbm, didx, ones_v, zbuf, acc):
    c = lax.axis_index("c")
    s = lax.axis_index("s")
    rbase = s * ROWS_PT
    pltpu.sync_copy(zeros_hbm, zbuf)
    pltpu.sync_copy(zbuf, acc.at[pl.ds(rbase, 512)])
    pltpu.sync_copy(zbuf.at[pl.ds(0, ROWS_PT - 512)],
                    acc.at[pl.ds(rbase + 512, ROWS_PT - 512)])
    pltpu.sync_copy(ones_hbm, ones_v)
    plsc.subcore_barrier()
    dst_t = dst_hbm.at[c]

    def block(b, carry):
        r0 = (s * BPT + b) * SUB
        pltpu.sync_copy(dst_t.at[pl.ds(r0, SUB)], didx)
        for j in range(SUB):
            pltpu.sync_copy(ones_v, acc.at[didx.at[j]], add=True)
        return carry

    lax.fori_loop(0, BPT, block, 0)
    plsc.subcore_barrier()
    obase = c * NPAD + rbase
    pltpu.sync_copy(acc.at[pl.ds(rbase, 512)], zbuf)
    pltpu.sync_copy(zbuf, out_hbm.at[pl.ds(obase, 512)])
    pltpu.sync_copy(acc.at[pl.ds(rbase + 512, ROWS_PT - 512)],
                    zbuf.at[pl.ds(0, ROWS_PT - 512)])
    pltpu.sync_copy(zbuf.at[pl.ds(0, ROWS_PT - 512)],
                    out_hbm.at[pl.ds(obase + 512, ROWS_PT - 512)])


@functools.partial(
    pl.kernel,
    out_type=jax.ShapeDtypeStruct((NC * NPAD, H), jnp.float32),
    mesh=_sc_mesh,
    scratch_types=[
        pltpu.VMEM((SUB, 128), jnp.int32),
        pltpu.VMEM((SUB, 128), jnp.int32),
        pltpu.VMEM((CHUNK, H), jnp.float32),
        pltpu.VMEM_SHARED((NPAD, H), jnp.float32),
        pltpu.SemaphoreType.DMA,
    ],
)
def _sc_segsum(src_hbm, dst_hbm, h_hbm, zeros_hbm, out_hbm,
               sidx, didx, rows, acc, sem):
    c = lax.axis_index("c")
    s = lax.axis_index("s")
    rbase = s * ROWS_PT
    pltpu.sync_copy(zeros_hbm, rows.at[pl.ds(0, 512)])
    pltpu.sync_copy(rows.at[pl.ds(0, 512)], acc.at[pl.ds(rbase, 512)])
    pltpu.sync_copy(rows.at[pl.ds(0, ROWS_PT - 512)],
                    acc.at[pl.ds(rbase + 512, ROWS_PT - 512)])
    plsc.subcore_barrier()
    src_t = src_hbm.at[c]
    dst_t = dst_hbm.at[c]

    def block(b, carry):
        r0 = (s * BPT + b) * SUB
        pltpu.sync_copy(src_t.at[pl.ds(r0, SUB)], sidx)
        pltpu.sync_copy(dst_t.at[pl.ds(r0, SUB)], didx)
        cps = [pltpu.async_copy(h_hbm.at[sidx.at[j]],
                                rows.at[pl.ds(j * 128, 128)], sem)
               for j in range(SUB)]
        for cp in cps:
            cp.wait()
        for j in range(SUB):
            pltpu.sync_copy(rows.at[pl.ds(j * 128, 128)],
                            acc.at[didx.at[j]], add=True)
        return carry

    lax.fori_loop(0, BPT, block, 0)
    plsc.subcore_barrier()
    obase = c * NPAD + rbase
    pltpu.sync_copy(acc.at[pl.ds(rbase, 512)], rows.at[pl.ds(0, 512)])
    pltpu.sync_copy(rows.at[pl.ds(0, 512)], out_hbm.at[pl.ds(obase, 512)])
    pltpu.sync_copy(acc.at[pl.ds(rbase + 512, ROWS_PT - 512)],
                    rows.at[pl.ds(0, ROWS_PT - 512)])
    pltpu.sync_copy(rows.at[pl.ds(0, ROWS_PT - 512)],
                    out_hbm.at[pl.ds(obase + 512, ROWS_PT - 512)])


def _ln(z, g, b):
    m = jnp.mean(z, axis=-1, keepdims=True)
    v = jnp.mean((z - m) ** 2, axis=-1, keepdims=True)
    return (z - m) / jnp.sqrt(v + 1e-5) * g + b


def _enc_body(xu_ref, xi_ref, w1_ref, b1_ref, w2_ref, b2_ref, g_ref, b_ref,
              out_ref):
    for t, x_ref in ((0, xu_ref), (1, xi_ref)):
        z = jnp.maximum(
            jnp.dot(x_ref[...], w1_ref[t], preferred_element_type=jnp.float32)
            + b1_ref[t][None, :], 0.0)
        z = jnp.dot(z, w2_ref[t], preferred_element_type=jnp.float32) \
            + b2_ref[t][None, :]
        out_ref[t] = _ln(z, g_ref[t][None, :], b_ref[t][None, :])


def _layer_body(h_ref, ssum_ref, cnt_ref, wl_ref, bl_ref, wr_ref, g_ref,
                b_ref, out_ref):
    for t in range(2):
        e = 1 - t
        cnt = jnp.maximum(cnt_ref[e, :, 0:1], 1.0)
        mean = ssum_ref[e] / cnt
        h_t = h_ref[t]
        upd = jnp.dot(mean, wl_ref[e], preferred_element_type=jnp.float32) \
            + bl_ref[e][None, :] \
            + jnp.dot(h_t, wr_ref[e], preferred_element_type=jnp.float32)
        nt = _ln(upd + h_t, g_ref[t][None, :], b_ref[t][None, :])
        out_ref[t] = jnp.maximum(nt, 0.0)


def _pool_body(h_ref, bu_ref, bi_ref, w1_ref, b1_ref, w2_ref, b2_ref,
               pu_ref, pi_ref, ge_ref, gs_ref, psum, pcnt):
    i = pl.program_id(0)

    @pl.when(i == 0)
    def _():
        psum[...] = jnp.zeros_like(psum)
        pcnt[...] = jnp.zeros_like(pcnt)

    for t, bref in ((0, bu_ref), (1, bi_ref)):
        bb = bref[0, 0, :]
        mask = (bb[:, None] ==
                lax.broadcasted_iota(jnp.int32, (BLK, NG), 1)
                ).astype(jnp.float32)
        psum[t] += lax.dot_general(mask, h_ref[t], (((0,), (0,)), ((), ())),
                                   preferred_element_type=jnp.float32)
        cnts = jnp.sum(mask, axis=0)
        pcnt[t] += jnp.broadcast_to(cnts[:, None], (NG, H))

    @pl.when(i == pl.num_programs(0) - 1)
    def _():
        pu = psum[0] / jnp.maximum(pcnt[0], 1.0)
        pi = psum[1] / jnp.maximum(pcnt[1], 1.0)
        pu_ref[...] = pu
        pi_ref[...] = pi
        ge = (pu + pi) * 0.5
        ge_ref[...] = ge
        hid = jnp.maximum(
            jnp.dot(ge, w1_ref[...], preferred_element_type=jnp.float32)
            + b1_ref[...], 0.0)
        gs_ref[...] = jnp.sum(hid * w2_ref[...], axis=-1, keepdims=True) \
            + b2_ref[...]


def _recon_body(h_ref, w_ref, b_ref, o0_ref, o1_ref):
    o0_ref[...] = jnp.dot(h_ref[0], w_ref[0],
                          preferred_element_type=jnp.float32) + b_ref[0][None, :]
    o1_ref[...] = jnp.dot(h_ref[1], w_ref[1],
                          preferred_element_type=jnp.float32) + b_ref[1][None, :]


def _row_spec(nrows=BLK, lead=None):
    if lead is None:
        return pl.BlockSpec((nrows, H), lambda i: (i, 0))
    return pl.BlockSpec((lead, nrows, H), lambda i: (0, i, 0))


def _full_spec(shape):
    nd = len(shape)
    return pl.BlockSpec(shape, lambda i: (0,) * nd)


def kernel(x_user, x_item, edge_index_u2i, edge_index_i2u, batch_user,
           batch_item, enc_W1, enc_b1, enc_W2, enc_b2, enc_ln_g, enc_ln_b,
           conv_Wl, conv_bl, conv_Wr, ln_g, ln_b, rec_W, rec_b, gs_W1, gs_b1,
           gs_W2, gs_b2):
    f32 = jnp.float32
    pad = EPAD - E

    def prep(ei, off):
        src = jnp.concatenate(
            [ei[0].astype(jnp.int32), jnp.zeros((pad,), jnp.int32)]) + off
        dst = jnp.concatenate(
            [ei[1].astype(jnp.int32), jnp.full((pad,), N, jnp.int32)])
        return src.reshape(EPAD // 128, 128), dst.reshape(EPAD // 128, 128)

    s0, d0 = prep(edge_index_u2i, 0)
    s1, d1 = prep(edge_index_i2u, N)
    src_all = jnp.stack([s0, s1])
    dst_all = jnp.stack([d0, d1])

    h = pl.pallas_call(
        _enc_body,
        grid=(GRID,),
        in_specs=[_row_spec(), _row_spec(),
                  _full_spec((2, H, H)), _full_spec((2, H)),
                  _full_spec((2, H, H)), _full_spec((2, H)),
                  _full_spec((2, H)), _full_spec((2, H))],
        out_specs=_row_spec(lead=2),
        out_shape=jax.ShapeDtypeStruct((2, N, H), f32),
    )(x_user, x_item, enc_W1, enc_b1, enc_W2, enc_b2, enc_ln_g, enc_ln_b)

    cnt = _sc_counts(dst_all,
                     jnp.ones((128, 16), f32),
                     jnp.zeros((512, 16), f32)).reshape(NC, NPAD, 16)

    zeros512 = jnp.zeros((512, H), f32)
    for l in range(L):
        ssum = _sc_segsum(src_all, dst_all, h.reshape(2 * N, H),
                          zeros512).reshape(NC, NPAD, H)
        h = pl.pallas_call(
            _layer_body,
            grid=(GRID,),
            in_specs=[_row_spec(lead=2), _row_spec(lead=2),
                      pl.BlockSpec((2, BLK, 16), lambda i: (0, i, 0)),
                      _full_spec((2, H, H)), _full_spec((2, H)),
                      _full_spec((2, H, H)), _full_spec((2, H)),
                      _full_spec((2, H))],
            out_specs=_row_spec(lead=2),
            out_shape=jax.ShapeDtypeStruct((2, N, H), f32),
        )(h, ssum, cnt, conv_Wl[l], conv_bl[l], conv_Wr[l], ln_g[l], ln_b[l])

    bu = batch_user.astype(jnp.int32).reshape(GRID, 1, BLK)
    bi = batch_item.astype(jnp.int32).reshape(GRID, 1, BLK)
    g16 = jax.ShapeDtypeStruct((NG, H), f32)
    pooled_u, pooled_i, graph_emb, gscore = pl.pallas_call(
        _pool_body,
        grid=(GRID,),
        in_specs=[_row_spec(lead=2),
                  pl.BlockSpec((1, 1, BLK), lambda i: (i, 0, 0)),
                  pl.BlockSpec((1, 1, BLK), lambda i: (i, 0, 0)),
                  _full_spec((H, H)), _full_spec((1, H)),
                  _full_spec((1, H)), _full_spec((1, 1))],
        out_specs=[_full_spec((NG, H)), _full_spec((NG, H)),
                   _full_spec((NG, H)), _full_spec((NG, 1))],
        out_shape=[g16, g16, g16, jax.ShapeDtypeStruct((NG, 1), f32)],
        scratch_shapes=[pltpu.VMEM((2, NG, H), f32),
                        pltpu.VMEM((2, NG, H), f32)],
    )(h, bu, bi, gs_W1, gs_b1.reshape(1, H), gs_W2.reshape(1, H),
      gs_b2.reshape(1, 1))

    recon0, recon1 = pl.pallas_call(
        _recon_body,
        grid=(GRID,),
        in_specs=[_row_spec(lead=2), _full_spec((2, H, H)),
                  _full_spec((2, H))],
        out_specs=[_row_spec(), _row_spec()],
        out_shape=[jax.ShapeDtypeStruct((N, H), f32)] * 2,
    )(h, rec_W, rec_b)

    return (h[0], h[1], graph_emb, gscore, recon0, recon1, pooled_u, pooled_i)


# speedup vs baseline: 3.8413x; 3.8413x over previous
import functools

import jax
import jax.numpy as jnp
from jax import lax
from jax.experimental import pallas as pl
from jax.experimental.pallas import tpu as pltpu
from jax.experimental.pallas import tpu_sc as plsc

N = 10000
H = 128
L = 3
NG = 16
E = 300000

NC = 2
NS = 16
CHUNK = 256
BPT = -(-E // (NS * CHUNK))
EPT = BPT * CHUNK
EPAD = NS * EPT
ROWS_PT = 632
NPAD = NS * ROWS_PT
BLK = 1000
GRID = N // BLK

def _sc_segsum_body(src_hbm, dst_hbm, h_hbm, zeros_hbm, out_hbm,
                    sidx0, sidx1, didx0, didx1, rows, acc, sem):
    c = lax.axis_index("c")
    s = lax.axis_index("s")
    rbase = s * ROWS_PT
    pltpu.sync_copy(zeros_hbm, rows)
    for off, ln in ((0, 256), (256, 256), (512, ROWS_PT - 512)):
        pltpu.sync_copy(rows.at[pl.ds(0, ln)], acc.at[pl.ds(rbase + off, ln)])
    plsc.subcore_barrier()
    src_t = src_hbm.at[c]
    dst_t = dst_hbm.at[c]

    def block(b, carry):
        e0 = s * EPT + b * CHUNK
        pltpu.sync_copy(src_t.at[pl.ds(e0, 128)], sidx0)
        pltpu.sync_copy(src_t.at[pl.ds(e0 + 128, 128)], sidx1)
        pltpu.sync_copy(dst_t.at[pl.ds(e0, 128)], didx0)
        pltpu.sync_copy(dst_t.at[pl.ds(e0 + 128, 128)], didx1)
        cp0 = pltpu.async_copy(h_hbm.at[sidx0], rows.at[pl.ds(0, 128)], sem)
        cp1 = pltpu.async_copy(h_hbm.at[sidx1], rows.at[pl.ds(128, 128)], sem)
        cp0.wait()
        cp1.wait()
        pltpu.sync_copy(rows.at[pl.ds(0, 128)], acc.at[didx0], add=True)
        pltpu.sync_copy(rows.at[pl.ds(128, 128)], acc.at[didx1], add=True)
        return carry

    lax.fori_loop(0, BPT, block, 0)
    plsc.subcore_barrier()
    obase = c * NPAD + rbase
    for off, ln in ((0, 256), (256, 256), (512, ROWS_PT - 512)):
        pltpu.sync_copy(acc.at[pl.ds(rbase + off, ln)], rows.at[pl.ds(0, ln)])
        pltpu.sync_copy(rows.at[pl.ds(0, ln)],
                        out_hbm.at[pl.ds(obase + off, ln)])


@functools.cache
def _sc_kernels():
    mesh = plsc.VectorSubcoreMesh(
        core_axis_name="c", subcore_axis_name="s",
        num_cores=NC, num_subcores=NS)
    segsum = pl.kernel(
        _sc_segsum_body,
        out_type=jax.ShapeDtypeStruct((NC * NPAD, H), jnp.float32),
        mesh=mesh,
        scratch_types=[
            pltpu.VMEM((128,), jnp.int32),
            pltpu.VMEM((128,), jnp.int32),
            pltpu.VMEM((128,), jnp.int32),
            pltpu.VMEM((128,), jnp.int32),
            pltpu.VMEM((CHUNK, H), jnp.float32),
            pltpu.VMEM_SHARED((NPAD, H), jnp.float32),
            pltpu.SemaphoreType.DMA,
        ],
    )
    return segsum


def _ln(z, g, b):
    m = jnp.mean(z, axis=-1, keepdims=True)
    v = jnp.mean((z - m) ** 2, axis=-1, keepdims=True)
    return (z - m) / jnp.sqrt(v + 1e-5) * g + b


def _enc_body(xu_ref, xi_ref, w1_ref, b1_ref, w2_ref, b2_ref, g_ref, b_ref,
              out_ref):
    for t, x_ref in ((0, xu_ref), (1, xi_ref)):
        z = jnp.maximum(
            jnp.dot(x_ref[...], w1_ref[t], preferred_element_type=jnp.float32)
            + b1_ref[t][None, :], 0.0)
        z = jnp.dot(z, w2_ref[t], preferred_element_type=jnp.float32) \
            + b2_ref[t][None, :]
        out_ref[t] = _ln(z, g_ref[t][None, :], b_ref[t][None, :])


def _layer_body(h_ref, ssum_ref, cnt_ref, wl_ref, bl_ref, wr_ref, g_ref,
                b_ref, out_ref):
    for t in range(2):
        e = 1 - t
        cnt = jnp.maximum(cnt_ref[e, :, 0:1], 1.0)
        mean = ssum_ref[e] / cnt
        h_t = h_ref[t]
        upd = jnp.dot(mean, wl_ref[e], preferred_element_type=jnp.float32) \
            + bl_ref[e][None, :] \
            + jnp.dot(h_t, wr_ref[e], preferred_element_type=jnp.float32)
        nt = _ln(upd + h_t, g_ref[t][None, :], b_ref[t][None, :])
        out_ref[t] = jnp.maximum(nt, 0.0)


def _pool_body(h_ref, bu_ref, bi_ref, w1_ref, b1_ref, w2_ref, b2_ref,
               pu_ref, pi_ref, ge_ref, gs_ref, psum, pcnt):
    i = pl.program_id(0)

    @pl.when(i == 0)
    def _():
        psum[...] = jnp.zeros_like(psum)
        pcnt[...] = jnp.zeros_like(pcnt)

    for t, bref in ((0, bu_ref), (1, bi_ref)):
        bb = bref[0, 0, :]
        mask = (bb[:, None] ==
                lax.broadcasted_iota(jnp.int32, (BLK, NG), 1)
                ).astype(jnp.float32)
        psum[t] += lax.dot_general(mask, h_ref[t], (((0,), (0,)), ((), ())),
                                   preferred_element_type=jnp.float32)
        cnts = jnp.sum(mask, axis=0)
        pcnt[t] += jnp.broadcast_to(cnts[:, None], (NG, H))

    @pl.when(i == pl.num_programs(0) - 1)
    def _():
        pu = psum[0] / jnp.maximum(pcnt[0], 1.0)
        pi = psum[1] / jnp.maximum(pcnt[1], 1.0)
        pu_ref[...] = pu
        pi_ref[...] = pi
        ge = (pu + pi) * 0.5
        ge_ref[...] = ge
        hid = jnp.maximum(
            jnp.dot(ge, w1_ref[...], preferred_element_type=jnp.float32)
            + b1_ref[...], 0.0)
        gs_ref[...] = jnp.sum(hid * w2_ref[...], axis=-1, keepdims=True) \
            + b2_ref[...]


def _recon_body(h_ref, w_ref, b_ref, o0_ref, o1_ref):
    o0_ref[...] = jnp.dot(h_ref[0], w_ref[0],
                          preferred_element_type=jnp.float32) + b_ref[0][None, :]
    o1_ref[...] = jnp.dot(h_ref[1], w_ref[1],
                          preferred_element_type=jnp.float32) + b_ref[1][None, :]


def _row_spec(nrows=BLK, lead=None):
    if lead is None:
        return pl.BlockSpec((nrows, H), lambda i: (i, 0))
    return pl.BlockSpec((lead, nrows, H), lambda i: (0, i, 0))


def _full_spec(shape):
    nd = len(shape)
    return pl.BlockSpec(shape, lambda i: (0,) * nd)


def kernel(x_user, x_item, edge_index_u2i, edge_index_i2u, batch_user,
           batch_item, enc_W1, enc_b1, enc_W2, enc_b2, enc_ln_g, enc_ln_b,
           conv_Wl, conv_bl, conv_Wr, ln_g, ln_b, rec_W, rec_b, gs_W1, gs_b1,
           gs_W2, gs_b2):
    f32 = jnp.float32
    pad = EPAD - E

    def prep(ei, off):
        src = jnp.concatenate(
            [ei[0].astype(jnp.int32), jnp.zeros((pad,), jnp.int32)]) + off
        dst = jnp.concatenate(
            [ei[1].astype(jnp.int32), jnp.full((pad,), N, jnp.int32)])
        return src, dst

    s0, d0 = prep(edge_index_u2i, 0)
    s1, d1 = prep(edge_index_i2u, N)
    src_all = jnp.stack([s0, s1])
    dst_all = jnp.stack([d0, d1])

    h = pl.pallas_call(
        _enc_body,
        grid=(GRID,),
        in_specs=[_row_spec(), _row_spec(),
                  _full_spec((2, H, H)), _full_spec((2, H)),
                  _full_spec((2, H, H)), _full_spec((2, H)),
                  _full_spec((2, H)), _full_spec((2, H))],
        out_specs=_row_spec(lead=2),
        out_shape=jax.ShapeDtypeStruct((2, N, H), f32),
    )(x_user, x_item, enc_W1, enc_b1, enc_W2, enc_b2, enc_ln_g, enc_ln_b)

    sc_segsum = _sc_kernels()
    zrows = jnp.zeros((CHUNK, H), f32)
    cnt = sc_segsum(dst_all, dst_all, jnp.ones((NPAD, H), f32),
                    zrows).reshape(NC, NPAD, H)

    h, cnt = lax.optimization_barrier((h, cnt))
    for l in range(L):
        ssum = sc_segsum(src_all, dst_all, h.reshape(2 * N, H),
                         zrows).reshape(NC, NPAD, H)
        h = pl.pallas_call(
            _layer_body,
            grid=(GRID,),
            in_specs=[_row_spec(lead=2), _row_spec(lead=2),
                      pl.BlockSpec((2, BLK, H), lambda i: (0, i, 0)),
                      _full_spec((2, H, H)), _full_spec((2, H)),
                      _full_spec((2, H, H)), _full_spec((2, H)),
                      _full_spec((2, H))],
            out_specs=_row_spec(lead=2),
            out_shape=jax.ShapeDtypeStruct((2, N, H), f32),
        )(h, ssum, cnt, conv_Wl[l], conv_bl[l], conv_Wr[l], ln_g[l], ln_b[l])

    bu = batch_user.astype(jnp.int32).reshape(GRID, 1, BLK)
    bi = batch_item.astype(jnp.int32).reshape(GRID, 1, BLK)
    g16 = jax.ShapeDtypeStruct((NG, H), f32)
    pooled_u, pooled_i, graph_emb, gscore = pl.pallas_call(
        _pool_body,
        grid=(GRID,),
        in_specs=[_row_spec(lead=2),
                  pl.BlockSpec((1, 1, BLK), lambda i: (i, 0, 0)),
                  pl.BlockSpec((1, 1, BLK), lambda i: (i, 0, 0)),
                  _full_spec((H, H)), _full_spec((1, H)),
                  _full_spec((1, H)), _full_spec((1, 1))],
        out_specs=[_full_spec((NG, H)), _full_spec((NG, H)),
                   _full_spec((NG, H)), _full_spec((NG, 1))],
        out_shape=[g16, g16, g16, jax.ShapeDtypeStruct((NG, 1), f32)],
        scratch_shapes=[pltpu.VMEM((2, NG, H), f32),
                        pltpu.VMEM((2, NG, H), f32)],
    )(h, bu, bi, gs_W1, gs_b1.reshape(1, H), gs_W2.reshape(1, H),
      gs_b2.reshape(1, 1))

    recon0, recon1 = pl.pallas_call(
        _recon_body,
        grid=(GRID,),
        in_specs=[_row_spec(lead=2), _full_spec((2, H, H)),
                  _full_spec((2, H))],
        out_specs=[_row_spec(), _row_spec()],
        out_shape=[jax.ShapeDtypeStruct((N, H), f32)] * 2,
    )(h, rec_W, rec_b)

    return (h[0], h[1], graph_emb, gscore, recon0, recon1, pooled_u, pooled_i)
